# baseline (speedup 1.0000x reference)
import functools

import jax
import jax.numpy as jnp
from jax import lax
from jax.experimental import pallas as pl
from jax.experimental.pallas import tpu as pltpu
from jax.experimental.pallas import tpu_sc as plsc

NC = 2
NS = 16
D = 16
CHUNK = 2000


def _segment_sum_sc(bonds, pairs_f32, zeros, n_nodes):
    n_edges = pairs_f32.shape[0]
    n_workers = NC * NS
    epw = n_edges // n_workers
    n_chunks = epw // CHUNK
    n_pad = ((n_nodes + NS * 8 - 1) // (NS * 8)) * (NS * 8)
    rows_per_tile = n_pad // NS

    mesh = plsc.VectorSubcoreMesh(core_axis_name="c", subcore_axis_name="s",
                                  num_cores=NC, num_subcores=NS)

    @functools.partial(
        pl.kernel,
        out_type=jax.ShapeDtypeStruct((NC, n_pad, D), jnp.float32),
        mesh=mesh,
        compiler_params=pltpu.CompilerParams(use_tc_tiling_on_sc=False,
                                             needs_layout_passes=False),
        scratch_types=[
            pltpu.VMEM((CHUNK, 2), jnp.float32),
            pltpu.VMEM((CHUNK,), jnp.int32),
            pltpu.VMEM((CHUNK,), jnp.int32),
            pltpu.VMEM((CHUNK, D), jnp.float32),
            pltpu.VMEM_SHARED((n_pad, D), jnp.float32),
            pltpu.SemaphoreType.DMA,
        ],
    )
    def seg_kernel(bonds_hbm, pairs_hbm, zeros_hbm, out_hbm,
                   pairs_v, nbr_v, src_v, rows_v, acc_sh, sem):
        c = lax.axis_index("c")
        s = lax.axis_index("s")
        r0 = s * rows_per_tile
        pltpu.sync_copy(zeros_hbm, acc_sh.at[pl.ds(r0, rows_per_tile)])
        plsc.subcore_barrier()
        base0 = (c * NS + s) * epw
        lanes = lax.iota(jnp.int32, 16)
        col0 = jnp.zeros((16,), jnp.int32)
        col1 = jnp.ones((16,), jnp.int32)
        for k in range(n_chunks):
            base = base0 + k * CHUNK
            pltpu.sync_copy(pairs_hbm.at[pl.ds(base, CHUNK)], pairs_v)

            def deint(w, carry):
                rows = w * 16 + lanes
                src_v[pl.ds(w * 16, 16)] = plsc.bitcast(
                    plsc.load_gather(pairs_v, [rows, col0]), jnp.int32)
                nbr_v[pl.ds(w * 16, 16)] = plsc.bitcast(
                    plsc.load_gather(pairs_v, [rows, col1]), jnp.int32)
                return carry

            lax.fori_loop(0, CHUNK // 16, deint, 0)
            pltpu.async_copy(bonds_hbm.at[nbr_v], rows_v, sem).wait()
            pltpu.sync_copy(rows_v, acc_sh.at[src_v], add=True)
        plsc.subcore_barrier()
        pltpu.sync_copy(acc_sh.at[pl.ds(r0, rows_per_tile)],
                        out_hbm.at[c, pl.ds(r0, rows_per_tile)])

    return seg_kernel(bonds, pairs_f32, zeros)


def _dense_tc(atoms, p0, p1, kernel_w, bias2d, wn, wi):
    n = atoms.shape[0]
    hid = wn.shape[0]
    blk = 2000
    grid = n // blk

    def body(atoms_ref, p0_ref, p1_ref, kw_ref, bias_ref, wn_ref, wi_ref,
             out_ref):
        a = jnp.dot(atoms_ref[...], kw_ref[...],
                    preferred_element_type=jnp.float32) + bias_ref[...]
        bond = p0_ref[...] + p1_ref[...]
        rj = lax.broadcasted_iota(jnp.int32, (D, D * D), 0)
        ct = lax.broadcasted_iota(jnp.int32, (D, D * D), 1)
        t_mat = (ct % D == rj).astype(jnp.float32)
        cs = lax.broadcasted_iota(jnp.int32, (D * D, D), 0)
        ri = lax.broadcasted_iota(jnp.int32, (D * D, D), 1)
        s_mat = (cs // D == ri).astype(jnp.float32)
        t = jnp.dot(bond, t_mat, preferred_element_type=jnp.float32)
        agg = jnp.dot(a * t, s_mat, preferred_element_type=jnp.float32)
        nodes = lax.dot_general(agg, wn_ref[...], (((1,), (1,)), ((), ())),
                                preferred_element_type=jnp.float32)
        edges = jnp.maximum(
            lax.dot_general(bond, wi_ref[...], (((1,), (1,)), ((), ())),
                            preferred_element_type=jnp.float32), 0.0)
        out_ref[...] = jnp.maximum(nodes + edges, 0.0)

    return pl.pallas_call(
        body,
        grid=(grid,),
        in_specs=[
            pl.BlockSpec((blk, D), lambda g: (g, 0)),
            pl.BlockSpec((blk, D), lambda g: (g, 0)),
            pl.BlockSpec((blk, D), lambda g: (g, 0)),
            pl.BlockSpec((D, D * D), lambda g: (0, 0)),
            pl.BlockSpec((1, D * D), lambda g: (0, 0)),
            pl.BlockSpec((hid, D), lambda g: (0, 0)),
            pl.BlockSpec((hid, D), lambda g: (0, 0)),
        ],
        out_specs=pl.BlockSpec((blk, hid), lambda g: (g, 0)),
        out_shape=jax.ShapeDtypeStruct((n, hid), jnp.float32),
    )(atoms, p0, p1, kernel_w, bias2d, wn, wi)


def kernel(atoms, bonds, pairs, kernel, bias, weight_node, weight_node_inp):
    n = atoms.shape[0]
    n_pad = ((n + NS * 8 - 1) // (NS * 8)) * (NS * 8)
    zeros = jnp.zeros((n_pad // NS, D), jnp.float32)
    pairs_f32 = lax.bitcast_convert_type(pairs, jnp.float32)
    partials = _segment_sum_sc(bonds, pairs_f32, zeros, n)
    result = _dense_tc(atoms, partials[0, :n], partials[1, :n], kernel,
                       jnp.reshape(bias, (1, -1)), weight_node,
                       weight_node_inp)
    return (result, result, result, result)


# speedup vs baseline: 2.3069x; 2.3069x over previous
import functools

import jax
import jax.numpy as jnp
from jax import lax
from jax.experimental import pallas as pl
from jax.experimental.pallas import tpu as pltpu
from jax.experimental.pallas import tpu_sc as plsc

NC = 2
NS = 16
D = 16
CHUNK = 2000


def _segment_sum_sc(bonds, src, nbr, zeros, n_nodes):
    n_edges = src.shape[0]
    n_workers = NC * NS
    epw = n_edges // n_workers
    n_chunks = epw // CHUNK
    n_pad = ((n_nodes + NS * 8 - 1) // (NS * 8)) * (NS * 8)
    rows_per_tile = n_pad // NS

    mesh = plsc.VectorSubcoreMesh(core_axis_name="c", subcore_axis_name="s",
                                  num_cores=NC, num_subcores=NS)

    @functools.partial(
        pl.kernel,
        out_type=jax.ShapeDtypeStruct((NC, n_pad, D), jnp.float32),
        mesh=mesh,
        compiler_params=pltpu.CompilerParams(use_tc_tiling_on_sc=False,
                                             needs_layout_passes=False),
        scratch_types=[
            pltpu.VMEM((CHUNK,), jnp.int32),
            pltpu.VMEM((CHUNK,), jnp.int32),
            pltpu.VMEM((CHUNK, D), jnp.float32),
            pltpu.VMEM_SHARED((n_pad, D), jnp.float32),
            pltpu.SemaphoreType.DMA,
        ],
    )
    def seg_kernel(bonds_hbm, src_hbm, nbr_hbm, zeros_hbm, out_hbm,
                   nbr_v, src_v, rows_v, acc_sh, sem):
        c = lax.axis_index("c")
        s = lax.axis_index("s")
        r0 = s * rows_per_tile
        pltpu.sync_copy(zeros_hbm, acc_sh.at[pl.ds(r0, rows_per_tile)])
        plsc.subcore_barrier()
        base0 = (c * NS + s) * epw
        for k in range(n_chunks):
            base = base0 + k * CHUNK
            pltpu.sync_copy(nbr_hbm.at[pl.ds(base, CHUNK)], nbr_v)
            pltpu.sync_copy(src_hbm.at[pl.ds(base, CHUNK)], src_v)
            pltpu.async_copy(bonds_hbm.at[nbr_v], rows_v, sem).wait()
            pltpu.sync_copy(rows_v, acc_sh.at[src_v], add=True)
        plsc.subcore_barrier()
        pltpu.sync_copy(acc_sh.at[pl.ds(r0, rows_per_tile)],
                        out_hbm.at[c, pl.ds(r0, rows_per_tile)])

    return seg_kernel(bonds, src, nbr, zeros)


def _dense_tc(atoms, p0, p1, kernel_w, bias2d, wn, wi):
    n = atoms.shape[0]
    hid = wn.shape[0]
    blk = 2000
    grid = n // blk

    def body(atoms_ref, p0_ref, p1_ref, kw_ref, bias_ref, wn_ref, wi_ref,
             out_ref):
        a = jnp.dot(atoms_ref[...], kw_ref[...],
                    preferred_element_type=jnp.float32) + bias_ref[...]
        bond = p0_ref[...] + p1_ref[...]
        rj = lax.broadcasted_iota(jnp.int32, (D, D * D), 0)
        ct = lax.broadcasted_iota(jnp.int32, (D, D * D), 1)
        t_mat = (ct % D == rj).astype(jnp.float32)
        cs = lax.broadcasted_iota(jnp.int32, (D * D, D), 0)
        ri = lax.broadcasted_iota(jnp.int32, (D * D, D), 1)
        s_mat = (cs // D == ri).astype(jnp.float32)
        t = jnp.dot(bond, t_mat, preferred_element_type=jnp.float32)
        agg = jnp.dot(a * t, s_mat, preferred_element_type=jnp.float32)
        nodes = lax.dot_general(agg, wn_ref[...], (((1,), (1,)), ((), ())),
                                preferred_element_type=jnp.float32)
        edges = jnp.maximum(
            lax.dot_general(bond, wi_ref[...], (((1,), (1,)), ((), ())),
                            preferred_element_type=jnp.float32), 0.0)
        out_ref[...] = jnp.maximum(nodes + edges, 0.0)

    return pl.pallas_call(
        body,
        grid=(grid,),
        in_specs=[
            pl.BlockSpec((blk, D), lambda g: (g, 0)),
            pl.BlockSpec((blk, D), lambda g: (g, 0)),
            pl.BlockSpec((blk, D), lambda g: (g, 0)),
            pl.BlockSpec((D, D * D), lambda g: (0, 0)),
            pl.BlockSpec((1, D * D), lambda g: (0, 0)),
            pl.BlockSpec((hid, D), lambda g: (0, 0)),
            pl.BlockSpec((hid, D), lambda g: (0, 0)),
        ],
        out_specs=pl.BlockSpec((blk, hid), lambda g: (g, 0)),
        out_shape=jax.ShapeDtypeStruct((n, hid), jnp.float32),
    )(atoms, p0, p1, kernel_w, bias2d, wn, wi)


def kernel(atoms, bonds, pairs, kernel, bias, weight_node, weight_node_inp):
    n = atoms.shape[0]
    n_pad = ((n + NS * 8 - 1) // (NS * 8)) * (NS * 8)
    zeros = jnp.zeros((n_pad // NS, D), jnp.float32)
    pairs_t = jnp.transpose(pairs)
    src = pairs_t[0]
    nbr = pairs_t[1]
    partials = _segment_sum_sc(bonds, src, nbr, zeros, n)
    result = _dense_tc(atoms, partials[0, :n], partials[1, :n], kernel,
                       jnp.reshape(bias, (1, -1)), weight_node,
                       weight_node_inp)
    return (result, result, result, result)
